# baseline (speedup 1.0000x reference)
import functools

import jax
import jax.numpy as jnp
from jax import lax
from jax.experimental import pallas as pl
from jax.experimental.pallas import tpu as pltpu
from jax.experimental.pallas import tpu_sc as plsc

N = 16384
C_IN = 128
C_OUT = 256
K = 16
STRIDE = 4
M = N // STRIDE
EPS = 1e-5


_G_BLOCK = 512


def _g_body(feats_ref, gamma_ref, beta_ref, w_ref, b_ref, out_ref):
    x = feats_ref[...]
    mean = jnp.mean(x, axis=1, keepdims=True)
    xc = x - mean
    var = jnp.mean(xc * xc, axis=1, keepdims=True)
    normed = xc * lax.rsqrt(var + EPS) * gamma_ref[...] + beta_ref[...]
    out_ref[...] = (
        jnp.dot(normed, w_ref[...], preferred_element_type=jnp.float32)
        + b_ref[...]
    )


def _compute_g(feats, gamma, beta, w, b):
    return pl.pallas_call(
        _g_body,
        out_shape=jax.ShapeDtypeStruct((N, C_OUT), jnp.float32),
        grid=(N // _G_BLOCK,),
        in_specs=[
            pl.BlockSpec((_G_BLOCK, C_IN), lambda i: (i, 0)),
            pl.BlockSpec((1, C_IN), lambda i: (0, 0)),
            pl.BlockSpec((1, C_IN), lambda i: (0, 0)),
            pl.BlockSpec((C_IN, C_OUT), lambda i: (0, 0)),
            pl.BlockSpec((1, C_OUT), lambda i: (0, 0)),
        ],
        out_specs=pl.BlockSpec((_G_BLOCK, C_OUT), lambda i: (i, 0)),
    )(feats, gamma.reshape(1, C_IN), beta.reshape(1, C_IN), w, b.reshape(1, C_OUT))


_QCHUNK = 16
_NWORK = 32
_QPW = M // _NWORK
_LANES = 16


def _gather_max(g_hbm, idx_hbm, out_hbm, idx_v, rows_v, out_v, sem):
    wid = lax.axis_index("s") * 2 + lax.axis_index("c")

    def chunk_body(ci, carry):
        base_q = wid * _QPW + ci * _QCHUNK
        pltpu.sync_copy(idx_hbm.at[pl.ds(base_q * K, _QCHUNK * K)], idx_v)
        pltpu.async_copy(g_hbm.at[idx_v], rows_v, sem).wait()

        def q_body(q, c2):
            def col_body(cb, c3):
                col = pl.ds(cb * _LANES, _LANES)
                acc = rows_v[q * K, col]
                for k in range(1, K):
                    acc = jnp.maximum(acc, rows_v[q * K + k, col])
                out_v[q, col] = acc
                return c3

            return lax.fori_loop(0, C_OUT // _LANES, col_body, c2)

        lax.fori_loop(0, _QCHUNK, q_body, 0)
        pltpu.sync_copy(out_v, out_hbm.at[pl.ds(base_q, _QCHUNK)])
        return carry

    lax.fori_loop(0, _QPW // _QCHUNK, chunk_body, 0)


def _run_gather_max(g, idx_flat):
    mesh = plsc.VectorSubcoreMesh(core_axis_name="c", subcore_axis_name="s")
    fn = functools.partial(
        pl.kernel,
        mesh=mesh,
        out_type=jax.ShapeDtypeStruct((M, C_OUT), jnp.float32),
        scratch_types=[
            pltpu.VMEM((_QCHUNK * K,), jnp.int32),
            pltpu.VMEM((_QCHUNK * K, C_OUT), jnp.float32),
            pltpu.VMEM((_QCHUNK, C_OUT), jnp.float32),
            pltpu.SemaphoreType.DMA,
        ],
    )(_gather_max)
    return fn(g, idx_flat)


_QB = 512


def _knn_body(q_ref, ct_ref, idx_ref):
    q = q_ref[...]
    ct = ct_ref[...]
    qn = jnp.sum(q * q, axis=1, keepdims=True)
    cn = jnp.sum(ct * ct, axis=0, keepdims=True)
    d = (qn - 2.0 * jnp.dot(q, ct, preferred_element_type=jnp.float32)) + cn
    iota = lax.broadcasted_iota(jnp.int32, (_QB, N), 1)
    for r in range(K):
        m = jnp.min(d, axis=1, keepdims=True)
        sel = jnp.where(d == m, iota, N)
        idx = jnp.min(sel, axis=1)
        idx_ref[:, r] = idx
        d = jnp.where(iota == idx[:, None], jnp.inf, d)


_T = 6
_NCH = 128
_CL = N // _NCH


def _knn_cache_body(c_ref, qt_ref, idxt_ref, flag_ref):
    c = c_ref[...]
    qt = qt_ref[...]
    cn = jnp.sum(c * c, axis=1, keepdims=True)
    qn = jnp.sum(qt * qt, axis=0, keepdims=True)
    dT = (qn - 2.0 * jnp.dot(c, qt, preferred_element_type=jnp.float32)) + cn
    d3 = dT.reshape(_NCH, _CL, _QB)
    icc = lax.broadcasted_iota(jnp.int32, (_NCH, _CL, _QB), 1)
    io2 = lax.broadcasted_iota(jnp.int32, (_NCH, _QB), 0)
    mv = []
    ai = []
    dm = d3
    for t in range(_T):
        m3 = jnp.min(dm, axis=1)
        a3c = jnp.min(jnp.where(dm == m3[:, None, :], icc, _CL), axis=1)
        mv.append(m3)
        ai.append(io2 * _CL + a3c)
        if t < _T - 1:
            dm = jnp.where(icc == a3c[:, None, :], jnp.inf, dm)
    mv5 = mv[_T - 1]
    m_last = None
    for r in range(K):
        mt = mv[0]
        for t in range(1, _T):
            mt = jnp.minimum(mt, mv[t])
        m = jnp.min(mt, axis=0, keepdims=True)
        idx = jnp.full((_QB,), N, jnp.int32)
        for t in range(_T):
            idx = jnp.minimum(
                idx, jnp.min(jnp.where(mv[t] == m, ai[t], N), axis=0)
            )
        idxt_ref[r, :] = idx
        for t in range(_T):
            mv[t] = jnp.where(ai[t] == idx[None, :], jnp.inf, mv[t])
        m_last = m
    trig = jnp.any(mv5 <= m_last)
    flag_ref[...] = jnp.broadcast_to(trig.astype(jnp.int32), (1, 1, 1))


def _knn_exact(qpad, ctpad):
    return pl.pallas_call(
        _knn_body,
        out_shape=jax.ShapeDtypeStruct((M, K), jnp.int32),
        grid=(M // _QB,),
        in_specs=[
            pl.BlockSpec((_QB, 8), lambda i: (i, 0)),
            pl.BlockSpec((8, N), lambda i: (0, 0)),
        ],
        out_specs=pl.BlockSpec((_QB, K), lambda i: (i, 0)),
    )(qpad, ctpad)


def _knn_idx(coords):
    coords_down = coords[::STRIDE]
    qpad = jnp.pad(coords_down, ((0, 0), (0, 5)))
    ctpad = jnp.pad(coords.T, ((0, 5), (0, 0)))
    cpad = jnp.pad(coords, ((0, 0), (0, 5)))
    qtpad = jnp.pad(coords_down.T, ((0, 5), (0, 0)))
    nblk = M // _QB
    idxt, flags = pl.pallas_call(
        _knn_cache_body,
        out_shape=(
            jax.ShapeDtypeStruct((K, M), jnp.int32),
            jax.ShapeDtypeStruct((nblk, 1, 1), jnp.int32),
        ),
        grid=(nblk,),
        in_specs=[
            pl.BlockSpec((N, 8), lambda i: (0, 0)),
            pl.BlockSpec((8, _QB), lambda i: (0, i)),
        ],
        out_specs=(
            pl.BlockSpec((K, _QB), lambda i: (0, i)),
            pl.BlockSpec((1, 1, 1), lambda i: (i, 0, 0)),
        ),
    )(cpad, qtpad)
    return lax.cond(
        jnp.any(flags > 0),
        lambda: _knn_exact(qpad, ctpad),
        lambda: idxt.T,
    )


def kernel(coords, feats, gamma, beta, W, b):
    g = _compute_g(feats, gamma, beta, W, b)
    knn_idx = _knn_idx(coords)
    return _run_gather_max(g, knn_idx.reshape(-1))


# speedup vs baseline: 1.1209x; 1.1209x over previous
import functools

import jax
import jax.numpy as jnp
from jax import lax
from jax.experimental import pallas as pl
from jax.experimental.pallas import tpu as pltpu
from jax.experimental.pallas import tpu_sc as plsc

N = 16384
C_IN = 128
C_OUT = 256
K = 16
STRIDE = 4
M = N // STRIDE
EPS = 1e-5


_G_BLOCK = 512


def _g_body(feats_ref, gamma_ref, beta_ref, w_ref, b_ref, out_ref):
    x = feats_ref[...]
    mean = jnp.mean(x, axis=1, keepdims=True)
    xc = x - mean
    var = jnp.mean(xc * xc, axis=1, keepdims=True)
    normed = xc * lax.rsqrt(var + EPS) * gamma_ref[...] + beta_ref[...]
    out_ref[...] = (
        jnp.dot(normed, w_ref[...], preferred_element_type=jnp.float32)
        + b_ref[...]
    )


def _compute_g(feats, gamma, beta, w, b):
    return pl.pallas_call(
        _g_body,
        out_shape=jax.ShapeDtypeStruct((N, C_OUT), jnp.float32),
        grid=(N // _G_BLOCK,),
        in_specs=[
            pl.BlockSpec((_G_BLOCK, C_IN), lambda i: (i, 0)),
            pl.BlockSpec((1, C_IN), lambda i: (0, 0)),
            pl.BlockSpec((1, C_IN), lambda i: (0, 0)),
            pl.BlockSpec((C_IN, C_OUT), lambda i: (0, 0)),
            pl.BlockSpec((1, C_OUT), lambda i: (0, 0)),
        ],
        out_specs=pl.BlockSpec((_G_BLOCK, C_OUT), lambda i: (i, 0)),
    )(feats, gamma.reshape(1, C_IN), beta.reshape(1, C_IN), w, b.reshape(1, C_OUT))


_QCHUNK = 16
_NWORK = 32
_QPW = M // _NWORK
_LANES = 16


def _gather_max(g_hbm, idx_hbm, out_hbm, idx_v, rows_v, out_v, sem):
    wid = lax.axis_index("s") * 2 + lax.axis_index("c")

    def chunk_body(ci, carry):
        base_q = wid * _QPW + ci * _QCHUNK
        pltpu.sync_copy(idx_hbm.at[pl.ds(base_q * K, _QCHUNK * K)], idx_v)
        pltpu.async_copy(g_hbm.at[idx_v], rows_v, sem).wait()

        def q_body(q, c2):
            def col_body(cb, c3):
                col = pl.ds(cb * _LANES, _LANES)
                acc = rows_v[q * K, col]
                for k in range(1, K):
                    acc = jnp.maximum(acc, rows_v[q * K + k, col])
                out_v[q, col] = acc
                return c3

            return lax.fori_loop(0, C_OUT // _LANES, col_body, c2)

        lax.fori_loop(0, _QCHUNK, q_body, 0)
        pltpu.sync_copy(out_v, out_hbm.at[pl.ds(base_q, _QCHUNK)])
        return carry

    lax.fori_loop(0, _QPW // _QCHUNK, chunk_body, 0)


def _run_gather_max(g, idx_flat):
    mesh = plsc.VectorSubcoreMesh(core_axis_name="c", subcore_axis_name="s")
    fn = functools.partial(
        pl.kernel,
        mesh=mesh,
        out_type=jax.ShapeDtypeStruct((M, C_OUT), jnp.float32),
        scratch_types=[
            pltpu.VMEM((_QCHUNK * K,), jnp.int32),
            pltpu.VMEM((_QCHUNK * K, C_OUT), jnp.float32),
            pltpu.VMEM((_QCHUNK, C_OUT), jnp.float32),
            pltpu.SemaphoreType.DMA,
        ],
    )(_gather_max)
    return fn(g, idx_flat)


_QB = 128


def _knn_body(q_ref, ct_ref, idx_ref):
    q = q_ref[...]
    ct = ct_ref[...]
    qn = jnp.sum(q * q, axis=1, keepdims=True)
    cn = jnp.sum(ct * ct, axis=0, keepdims=True)
    d = (qn - 2.0 * jnp.dot(q, ct, preferred_element_type=jnp.float32)) + cn
    iota = lax.broadcasted_iota(jnp.int32, (_QB, N), 1)
    for r in range(K):
        m = jnp.min(d, axis=1, keepdims=True)
        sel = jnp.where(d == m, iota, N)
        idx = jnp.min(sel, axis=1)
        idx_ref[:, r] = idx
        d = jnp.where(iota == idx[:, None], jnp.inf, d)


_T = 6
_NCH = 128
_CL = N // _NCH


def _knn_cache_body(c_ref, qt_ref, idxt_ref, flag_ref):
    c = c_ref[...]
    qt = qt_ref[...]
    cn = jnp.sum(c * c, axis=1, keepdims=True)
    qn = jnp.sum(qt * qt, axis=0, keepdims=True)
    dT = (qn - 2.0 * jnp.dot(c, qt, preferred_element_type=jnp.float32)) + cn
    d3 = dT.reshape(_NCH, _CL, _QB)
    icc = lax.broadcasted_iota(jnp.int32, (_NCH, _CL, _QB), 1)
    io2 = lax.broadcasted_iota(jnp.int32, (_NCH, _QB), 0)
    mv = []
    ai = []
    dm = d3
    for t in range(_T):
        m3 = jnp.min(dm, axis=1)
        a3c = jnp.min(jnp.where(dm == m3[:, None, :], icc, _CL), axis=1)
        mv.append(m3)
        ai.append(io2 * _CL + a3c)
        if t < _T - 1:
            dm = jnp.where(icc == a3c[:, None, :], jnp.inf, dm)
    mv5 = mv[_T - 1]
    m_last = None
    for r in range(K):
        mt = mv[0]
        for t in range(1, _T):
            mt = jnp.minimum(mt, mv[t])
        m = jnp.min(mt, axis=0, keepdims=True)
        idx = jnp.full((_QB,), N, jnp.int32)
        for t in range(_T):
            idx = jnp.minimum(
                idx, jnp.min(jnp.where(mv[t] == m, ai[t], N), axis=0)
            )
        idxt_ref[r, :] = idx
        for t in range(_T):
            mv[t] = jnp.where(ai[t] == idx[None, :], jnp.inf, mv[t])
        m_last = m
    trig = jnp.any(mv5 <= m_last)
    flag_ref[...] = jnp.broadcast_to(trig.astype(jnp.int32), (1, 1, 1))


def _knn_exact(qpad, ctpad):
    return pl.pallas_call(
        _knn_body,
        out_shape=jax.ShapeDtypeStruct((M, K), jnp.int32),
        grid=(M // _QB,),
        in_specs=[
            pl.BlockSpec((_QB, 8), lambda i: (i, 0)),
            pl.BlockSpec((8, N), lambda i: (0, 0)),
        ],
        out_specs=pl.BlockSpec((_QB, K), lambda i: (i, 0)),
    )(qpad, ctpad)


def _knn_idx(coords):
    coords_down = coords[::STRIDE]
    qpad = jnp.pad(coords_down, ((0, 0), (0, 5)))
    ctpad = jnp.pad(coords.T, ((0, 5), (0, 0)))
    cpad = jnp.pad(coords, ((0, 0), (0, 5)))
    qtpad = jnp.pad(coords_down.T, ((0, 5), (0, 0)))
    nblk = M // _QB
    idxt, flags = pl.pallas_call(
        _knn_cache_body,
        out_shape=(
            jax.ShapeDtypeStruct((K, M), jnp.int32),
            jax.ShapeDtypeStruct((nblk, 1, 1), jnp.int32),
        ),
        grid=(nblk,),
        in_specs=[
            pl.BlockSpec((N, 8), lambda i: (0, 0)),
            pl.BlockSpec((8, _QB), lambda i: (0, i)),
        ],
        out_specs=(
            pl.BlockSpec((K, _QB), lambda i: (0, i)),
            pl.BlockSpec((1, 1, 1), lambda i: (i, 0, 0)),
        ),
    )(cpad, qtpad)
    return lax.cond(
        jnp.any(flags > 0),
        lambda: _knn_exact(qpad, ctpad),
        lambda: idxt.T,
    )


def kernel(coords, feats, gamma, beta, W, b):
    g = _compute_g(feats, gamma, beta, W, b)
    knn_idx = _knn_idx(coords)
    return _run_gather_max(g, knn_idx.reshape(-1))


# speedup vs baseline: 1.1298x; 1.0079x over previous
import functools

import jax
import jax.numpy as jnp
from jax import lax
from jax.experimental import pallas as pl
from jax.experimental.pallas import tpu as pltpu
from jax.experimental.pallas import tpu_sc as plsc

N = 16384
C_IN = 128
C_OUT = 256
K = 16
STRIDE = 4
M = N // STRIDE
EPS = 1e-5


_G_BLOCK = 512


def _g_body(feats_ref, gamma_ref, beta_ref, w_ref, b_ref, out_ref):
    x = feats_ref[...]
    mean = jnp.mean(x, axis=1, keepdims=True)
    xc = x - mean
    var = jnp.mean(xc * xc, axis=1, keepdims=True)
    normed = xc * lax.rsqrt(var + EPS) * gamma_ref[...] + beta_ref[...]
    out_ref[...] = (
        jnp.dot(normed, w_ref[...], preferred_element_type=jnp.float32)
        + b_ref[...]
    )


def _compute_g(feats, gamma, beta, w, b):
    return pl.pallas_call(
        _g_body,
        out_shape=jax.ShapeDtypeStruct((N, C_OUT), jnp.float32),
        grid=(N // _G_BLOCK,),
        in_specs=[
            pl.BlockSpec((_G_BLOCK, C_IN), lambda i: (i, 0)),
            pl.BlockSpec((1, C_IN), lambda i: (0, 0)),
            pl.BlockSpec((1, C_IN), lambda i: (0, 0)),
            pl.BlockSpec((C_IN, C_OUT), lambda i: (0, 0)),
            pl.BlockSpec((1, C_OUT), lambda i: (0, 0)),
        ],
        out_specs=pl.BlockSpec((_G_BLOCK, C_OUT), lambda i: (i, 0)),
    )(feats, gamma.reshape(1, C_IN), beta.reshape(1, C_IN), w, b.reshape(1, C_OUT))


_QCHUNK = 16
_NWORK = 32
_QPW = M // _NWORK
_LANES = 16


def _gather_max(g_hbm, idx_hbm, out_hbm, idx_v, rows_v, out_v, sem, n_q=M):
    wid = lax.axis_index("s") * 2 + lax.axis_index("c")
    qpw = n_q // _NWORK

    def chunk_body(ci, carry):
        base_q = wid * qpw + ci * _QCHUNK
        pltpu.sync_copy(idx_hbm.at[pl.ds(base_q * K, _QCHUNK * K)], idx_v)
        pltpu.async_copy(g_hbm.at[idx_v], rows_v, sem).wait()

        def q_body(q, c2):
            def col_body(cb, c3):
                col = pl.ds(cb * _LANES, _LANES)
                acc = rows_v[q * K, col]
                for k in range(1, K):
                    acc = jnp.maximum(acc, rows_v[q * K + k, col])
                out_v[q, col] = acc
                return c3

            return lax.fori_loop(0, C_OUT // _LANES, col_body, c2)

        lax.fori_loop(0, _QCHUNK, q_body, 0)
        pltpu.sync_copy(out_v, out_hbm.at[pl.ds(base_q, _QCHUNK)])
        return carry

    lax.fori_loop(0, qpw // _QCHUNK, chunk_body, 0)


def _run_gather_max(g, idx_flat, n_q=M):
    mesh = plsc.VectorSubcoreMesh(core_axis_name="c", subcore_axis_name="s")
    fn = functools.partial(
        pl.kernel,
        mesh=mesh,
        out_type=jax.ShapeDtypeStruct((n_q, C_OUT), jnp.float32),
        scratch_types=[
            pltpu.VMEM((_QCHUNK * K,), jnp.int32),
            pltpu.VMEM((_QCHUNK * K, C_OUT), jnp.float32),
            pltpu.VMEM((_QCHUNK, C_OUT), jnp.float32),
            pltpu.SemaphoreType.DMA,
        ],
    )(functools.partial(_gather_max, n_q=n_q))
    return fn(g, idx_flat)


_QB = 256


def _knn_body(q_ref, ct_ref, idx_ref):
    q = q_ref[...]
    ct = ct_ref[...]
    qn = jnp.sum(q * q, axis=1, keepdims=True)
    cn = jnp.sum(ct * ct, axis=0, keepdims=True)
    d = (qn - 2.0 * jnp.dot(q, ct, preferred_element_type=jnp.float32)) + cn
    iota = lax.broadcasted_iota(jnp.int32, (_QB, N), 1)
    for r in range(K):
        m = jnp.min(d, axis=1, keepdims=True)
        sel = jnp.where(d == m, iota, N)
        idx = jnp.min(sel, axis=1)
        idx_ref[:, r] = idx
        d = jnp.where(iota == idx[:, None], jnp.inf, d)


_T = 6
_NCH = 128
_CL = N // _NCH


def _knn_cache_body(c_ref, qt_ref, idxt_ref, flag_ref):
    c = c_ref[...]
    qt = qt_ref[...]
    cn = jnp.sum(c * c, axis=1, keepdims=True)
    qn = jnp.sum(qt * qt, axis=0, keepdims=True)
    dT = (qn - 2.0 * jnp.dot(c, qt, preferred_element_type=jnp.float32)) + cn
    d3 = dT.reshape(_NCH, _CL, _QB)
    icc = lax.broadcasted_iota(jnp.int32, (_NCH, _CL, _QB), 1)
    io2 = lax.broadcasted_iota(jnp.int32, (_NCH, _QB), 0)
    mv = []
    ai = []
    dm = d3
    for t in range(_T):
        m3 = jnp.min(dm, axis=1)
        a3c = jnp.min(jnp.where(dm == m3[:, None, :], icc, _CL), axis=1)
        mv.append(m3)
        ai.append(io2 * _CL + a3c)
        if t < _T - 1:
            dm = jnp.where(icc == a3c[:, None, :], jnp.inf, dm)
    mv5 = mv[_T - 1]
    m_last = None
    for r in range(K):
        mt = mv[0]
        for t in range(1, _T):
            mt = jnp.minimum(mt, mv[t])
        m = jnp.min(mt, axis=0, keepdims=True)
        idx = jnp.full((_QB,), N, jnp.int32)
        for t in range(_T):
            idx = jnp.minimum(
                idx, jnp.min(jnp.where(mv[t] == m, ai[t], N), axis=0)
            )
        idxt_ref[r, :] = idx
        for t in range(_T):
            mv[t] = jnp.where(ai[t] == idx[None, :], jnp.inf, mv[t])
        m_last = m
    trig = jnp.any(mv5 <= m_last)
    flag_ref[...] = jnp.broadcast_to(trig.astype(jnp.int32), (1, 1, 1))


def _knn_exact(qpad, ctpad):
    n_q = qpad.shape[0]
    return pl.pallas_call(
        _knn_body,
        out_shape=jax.ShapeDtypeStruct((n_q, K), jnp.int32),
        grid=(n_q // _QB,),
        in_specs=[
            pl.BlockSpec((_QB, 8), lambda i: (i, 0)),
            pl.BlockSpec((8, N), lambda i: (0, 0)),
        ],
        out_specs=pl.BlockSpec((_QB, K), lambda i: (i, 0)),
    )(qpad, ctpad)


def _knn_idx(cpad, qtpad, qpad, ctpad):
    n_q = qtpad.shape[1]
    nblk = n_q // _QB
    idxt, flags = pl.pallas_call(
        _knn_cache_body,
        out_shape=(
            jax.ShapeDtypeStruct((K, n_q), jnp.int32),
            jax.ShapeDtypeStruct((nblk, 1, 1), jnp.int32),
        ),
        grid=(nblk,),
        in_specs=[
            pl.BlockSpec((N, 8), lambda i: (0, 0)),
            pl.BlockSpec((8, _QB), lambda i: (0, i)),
        ],
        out_specs=(
            pl.BlockSpec((K, _QB), lambda i: (0, i)),
            pl.BlockSpec((1, 1, 1), lambda i: (i, 0, 0)),
        ),
    )(cpad, qtpad)
    return lax.cond(
        jnp.any(flags > 0),
        lambda: _knn_exact(qpad, ctpad),
        lambda: idxt.T,
    )


def kernel(coords, feats, gamma, beta, W, b):
    g = _compute_g(feats, gamma, beta, W, b)
    coords_down = coords[::STRIDE]
    qpad = jnp.pad(coords_down, ((0, 0), (0, 5)))
    ctpad = jnp.pad(coords.T, ((0, 5), (0, 0)))
    cpad = jnp.pad(coords, ((0, 0), (0, 5)))
    qtpad = jnp.pad(coords_down.T, ((0, 5), (0, 0)))
    mh = M // 2
    outs = []
    for h in range(2):
        sl = slice(h * mh, (h + 1) * mh)
        idx_h = _knn_idx(cpad, qtpad[:, sl], qpad[sl], ctpad)
        outs.append(_run_gather_max(g, idx_h.reshape(-1), mh))
    return jnp.concatenate(outs, axis=0)
